# Initial kernel scaffold; baseline (speedup 1.0000x reference)
import jax
import jax.numpy as jnp
from jax.experimental import pallas as pl


def kernel(x, condition_tensors):
    raise NotImplementedError("write your pallas kernel here")


# speedup vs baseline: 183.4026x; 183.4026x over previous
import functools

import jax
import jax.numpy as jnp
from jax import lax
from jax.experimental import pallas as pl
from jax.experimental.pallas import tpu as pltpu
from jax.experimental.pallas import tpu_sc as plsc

_NC = 2
_NS = 16
_NW = _NC * _NS
_L = 16
_K = 128

_N = 500000
_CH = 15632
_NP = _CH * _NW

_SIGN = jnp.int32(-2147483648)


def _ord32(v):
    b0 = jnp.left_shift(jnp.bitwise_and(v, 0xFF), 24)
    b1 = jnp.left_shift(jnp.bitwise_and(v, 0xFF00), 8)
    b2 = jnp.bitwise_and(lax.shift_right_logical(v, 8), 0xFF00)
    b3 = jnp.bitwise_and(lax.shift_right_logical(v, 24), 0xFF)
    return jnp.bitwise_xor(b0 | b1 | b2 | b3, _SIGN)


@functools.partial(
    pl.kernel,
    out_type=jax.ShapeDtypeStruct((_NP,), jnp.int32),
    mesh=plsc.VectorSubcoreMesh(core_axis_name="c", subcore_axis_name="s"),
    compiler_params=pltpu.CompilerParams(needs_layout_passes=False),
    scratch_types=[
        pltpu.VMEM((_CH,), jnp.int32),
        pltpu.VMEM((_CH,), jnp.int32),
        pltpu.VMEM((_K,), jnp.int32),
    ],
)
def _sc_lookup(x_hbm, keys_hbm, out_hbm, xv, ov, kv):
    wid = lax.axis_index("s") * _NC + lax.axis_index("c")
    base = wid * _CH
    pltpu.sync_copy(keys_hbm, kv)
    pltpu.sync_copy(x_hbm.at[pl.ds(base, _CH)], xv)
    for j in range(_K // _L):
        kv[pl.ds(j * _L, _L)] = _ord32(kv[pl.ds(j * _L, _L)])

    def body(i, carry):
        xs = _ord32(xv[pl.ds(i * _L, _L)])
        pos = jnp.zeros((_L,), jnp.int32)
        for step in (64, 32, 16, 8, 4, 2, 1):
            probe = pos + (step - 1)
            kk = plsc.load_gather(kv, [probe])
            pos = pos + jnp.where(kk < xs, step, 0)
        ov[pl.ds(i * _L, _L)] = pos
        return carry

    lax.fori_loop(0, _CH // _L, body, 0)
    pltpu.sync_copy(ov, out_hbm.at[pl.ds(base, _CH)])


def kernel(x, condition_tensors):
    keys = condition_tensors.reshape(_K)
    xp = jnp.concatenate([x, jnp.zeros((_NP - _N,), jnp.int32)])
    return _sc_lookup(xp, keys)[:_N]


# speedup vs baseline: 331.9207x; 1.8098x over previous
import functools

import jax
import jax.numpy as jnp
from jax import lax
from jax.experimental import pallas as pl
from jax.experimental.pallas import tpu as pltpu
from jax.experimental.pallas import tpu_sc as plsc

_NC = 2
_NS = 16
_NW = _NC * _NS
_L = 16
_K = 128

_N = 500000
_U = 4
_CH = 15680
_NP = _CH * _NW

_SIGN = jnp.int32(-2147483648)


def _ord32(v):
    b0 = jnp.left_shift(jnp.bitwise_and(v, 0xFF), 24)
    b1 = jnp.left_shift(jnp.bitwise_and(v, 0xFF00), 8)
    b2 = jnp.bitwise_and(lax.shift_right_logical(v, 8), 0xFF00)
    b3 = jnp.bitwise_and(lax.shift_right_logical(v, 24), 0xFF)
    return jnp.bitwise_xor(b0 | b1 | b2 | b3, _SIGN)


@functools.partial(
    pl.kernel,
    out_type=jax.ShapeDtypeStruct((_NP,), jnp.int32),
    mesh=plsc.VectorSubcoreMesh(core_axis_name="c", subcore_axis_name="s"),
    compiler_params=pltpu.CompilerParams(needs_layout_passes=False),
    scratch_types=[
        pltpu.VMEM((_CH,), jnp.int32),
        pltpu.VMEM((_CH,), jnp.int32),
        pltpu.VMEM((_K,), jnp.int32),
    ],
)
def _sc_lookup(x_hbm, keys_hbm, out_hbm, xv, ov, kv):
    wid = lax.axis_index("s") * _NC + lax.axis_index("c")
    base = wid * _CH
    pltpu.sync_copy(keys_hbm, kv)
    pltpu.sync_copy(x_hbm.at[pl.ds(base, _CH)], xv)
    for j in range(_K // _L):
        kv[pl.ds(j * _L, _L)] = _ord32(kv[pl.ds(j * _L, _L)])

    def body(i, carry):
        b = i * (_U * _L)
        xs = [_ord32(xv[pl.ds(b + k * _L, _L)]) for k in range(_U)]
        pos = [jnp.zeros((_L,), jnp.int32) for _ in range(_U)]
        for step in (64, 32, 16, 8, 4, 2, 1):
            for k in range(_U):
                kk = plsc.load_gather(kv, [pos[k] + (step - 1)])
                pos[k] = pos[k] + jnp.where(kk < xs[k], step, 0)
        for k in range(_U):
            ov[pl.ds(b + k * _L, _L)] = pos[k]
        return carry

    lax.fori_loop(0, _CH // (_U * _L), body, 0)
    pltpu.sync_copy(ov, out_hbm.at[pl.ds(base, _CH)])


def kernel(x, condition_tensors):
    keys = condition_tensors.reshape(_K)
    xp = jnp.concatenate([x, jnp.zeros((_NP - _N,), jnp.int32)])
    return _sc_lookup(xp, keys)[:_N]


# speedup vs baseline: 389.2666x; 1.1728x over previous
import functools

import jax
import jax.numpy as jnp
from jax import lax
from jax.experimental import pallas as pl
from jax.experimental.pallas import tpu as pltpu
from jax.experimental.pallas import tpu_sc as plsc

_NC = 2
_NS = 16
_NW = _NC * _NS
_L = 16
_K = 128

_N = 500000
_U = 8
_CH = 15744
_NP = _CH * _NW

_SIGN = jnp.int32(-2147483648)


def _ord32(v):
    b0 = jnp.left_shift(jnp.bitwise_and(v, 0xFF), 24)
    b1 = jnp.left_shift(jnp.bitwise_and(v, 0xFF00), 8)
    b2 = jnp.bitwise_and(lax.shift_right_logical(v, 8), 0xFF00)
    b3 = jnp.bitwise_and(lax.shift_right_logical(v, 24), 0xFF)
    return jnp.bitwise_xor(b0 | b1 | b2 | b3, _SIGN)


@functools.partial(
    pl.kernel,
    out_type=jax.ShapeDtypeStruct((_NP,), jnp.int32),
    mesh=plsc.VectorSubcoreMesh(core_axis_name="c", subcore_axis_name="s"),
    compiler_params=pltpu.CompilerParams(needs_layout_passes=False),
    scratch_types=[
        pltpu.VMEM((_CH,), jnp.int32),
        pltpu.VMEM((_CH,), jnp.int32),
        pltpu.VMEM((_K,), jnp.int32),
    ],
)
def _sc_lookup(x_hbm, keys_hbm, out_hbm, xv, ov, kv):
    wid = lax.axis_index("s") * _NC + lax.axis_index("c")
    base = wid * _CH
    pltpu.sync_copy(keys_hbm, kv)
    pltpu.sync_copy(x_hbm.at[pl.ds(base, _CH)], xv)
    for j in range(_K // _L):
        kv[pl.ds(j * _L, _L)] = _ord32(kv[pl.ds(j * _L, _L)])

    def body(i, carry):
        b = i * (_U * _L)
        xs = [_ord32(xv[pl.ds(b + k * _L, _L)]) for k in range(_U)]
        pos = [jnp.zeros((_L,), jnp.int32) for _ in range(_U)]
        for step in (64, 32, 16, 8, 4, 2, 1):
            for k in range(_U):
                kk = plsc.load_gather(kv, [pos[k] + (step - 1)])
                pos[k] = pos[k] + jnp.where(kk < xs[k], step, 0)
        for k in range(_U):
            ov[pl.ds(b + k * _L, _L)] = pos[k]
        return carry

    lax.fori_loop(0, _CH // (_U * _L), body, 0)
    pltpu.sync_copy(ov, out_hbm.at[pl.ds(base, _CH)])


def kernel(x, condition_tensors):
    keys = condition_tensors.reshape(_K)
    xp = jnp.concatenate([x, jnp.zeros((_NP - _N,), jnp.int32)])
    return _sc_lookup(xp, keys)[:_N]


# speedup vs baseline: 439.1005x; 1.1280x over previous
import functools

import jax
import jax.numpy as jnp
from jax import lax
from jax.experimental import pallas as pl
from jax.experimental.pallas import tpu as pltpu
from jax.experimental.pallas import tpu_sc as plsc

_NC = 2
_NS = 16
_NW = _NC * _NS
_L = 16
_K = 128

_N = 500000
_U = 8
_CH = 15744

_SIGN = jnp.int32(-2147483648)


def _ord32(v):
    b0 = jnp.left_shift(jnp.bitwise_and(v, 0xFF), 24)
    b1 = jnp.left_shift(jnp.bitwise_and(v, 0xFF00), 8)
    b2 = jnp.bitwise_and(lax.shift_right_logical(v, 8), 0xFF00)
    b3 = jnp.bitwise_and(lax.shift_right_logical(v, 24), 0xFF)
    return jnp.bitwise_xor(b0 | b1 | b2 | b3, _SIGN)


@functools.partial(
    pl.kernel,
    out_type=jax.ShapeDtypeStruct((_N,), jnp.int32),
    mesh=plsc.VectorSubcoreMesh(core_axis_name="c", subcore_axis_name="s"),
    compiler_params=pltpu.CompilerParams(needs_layout_passes=False),
    scratch_types=[
        pltpu.VMEM((_CH,), jnp.int32),
        pltpu.VMEM((_CH,), jnp.int32),
        pltpu.VMEM((_K,), jnp.int32),
    ],
)
def _sc_lookup(x_hbm, keys_hbm, out_hbm, xv, ov, kv):
    wid = lax.axis_index("s") * _NC + lax.axis_index("c")
    base = jnp.minimum(wid * _CH, _N - _CH)
    pltpu.sync_copy(keys_hbm, kv)
    pltpu.sync_copy(x_hbm.at[pl.ds(base, _CH)], xv)
    for j in range(_K // _L):
        kv[pl.ds(j * _L, _L)] = _ord32(kv[pl.ds(j * _L, _L)])

    def body(i, carry):
        b = i * (_U * _L)
        xs = [_ord32(xv[pl.ds(b + k * _L, _L)]) for k in range(_U)]
        pos = [jnp.zeros((_L,), jnp.int32) for _ in range(_U)]
        for step in (64, 32, 16, 8, 4, 2, 1):
            for k in range(_U):
                kk = plsc.load_gather(kv, [pos[k] + (step - 1)])
                pos[k] = pos[k] + jnp.where(kk < xs[k], step, 0)
        for k in range(_U):
            ov[pl.ds(b + k * _L, _L)] = pos[k]
        return carry

    lax.fori_loop(0, _CH // (_U * _L), body, 0)
    pltpu.sync_copy(ov, out_hbm.at[pl.ds(base, _CH)])


def kernel(x, condition_tensors):
    return _sc_lookup(x, condition_tensors.reshape(_K))
